# baseline (speedup 1.0000x reference)
import functools

import jax
import jax.numpy as jnp
from jax import lax
from jax.experimental import pallas as pl
from jax.experimental.pallas import tpu as pltpu
from jax.experimental.pallas import tpu_sc as plsc

_NC = 2
_NS = 16
_NW = _NC * _NS
_CHUNK = 64


def _embed_lookup(ids, pids, word_table, pos_table):
    b, s = ids.shape
    d = word_table.shape[1]
    n_total = b * s
    cpw = n_total // (_NW * _CHUNK)
    spw = cpw * _CHUNK
    wpb = s // spw
    mesh = plsc.VectorSubcoreMesh(core_axis_name="c", subcore_axis_name="s")

    @functools.partial(
        pl.kernel,
        out_type=jax.ShapeDtypeStruct((b, s, d), jnp.float32),
        mesh=mesh,
        scratch_types=[
            pltpu.VMEM((spw,), jnp.int32),
            pltpu.VMEM((spw,), jnp.int32),
            pltpu.VMEM((cpw, _CHUNK, d), jnp.float32),
            pltpu.VMEM_SHARED((pos_table.shape[0], d), jnp.float32),
            pltpu.SemaphoreType.DMA,
            pltpu.SemaphoreType.DMA,
            pltpu.SemaphoreType.DMA,
            pltpu.SemaphoreType.DMA,
            pltpu.SemaphoreType.DMA,
        ],
    )
    def k(ids_hbm, pids_hbm, wt_hbm, pt_hbm, out_hbm,
          widx, pidx, wrows, pt_sp, gsem, psem, isem, ssem, stsem):
        sid = lax.axis_index("s")
        wid = sid * _NC + lax.axis_index("c")
        row = wid // wpb
        col0 = (wid % wpb) * spw
        ic1 = pltpu.async_copy(ids_hbm.at[row, pl.ds(col0, spw)], widx, isem)
        ic2 = pltpu.async_copy(pids_hbm.at[row, pl.ds(col0, spw)], pidx, isem)
        prows_per_tile = pos_table.shape[0] // _NS
        pslice = pl.ds(sid * prows_per_tile, prows_per_tile)
        stage = pltpu.async_copy(pt_hbm.at[pslice], pt_sp.at[pslice], stsem)
        ic1.wait()

        @pl.loop(0, cpw)
        def _word(j):
            js = pl.ds(j * _CHUNK, _CHUNK)
            pltpu.async_copy(wt_hbm.at[widx.at[js]], wrows.at[j], gsem)

        ic2.wait()
        stage.wait()
        plsc.subcore_barrier()

        @pl.loop(0, cpw)
        def _pos(j):
            pltpu.make_async_copy(wt_hbm.at[pl.ds(0, _CHUNK)], wrows.at[j], gsem).wait()
            js = pl.ds(j * _CHUNK, _CHUNK)
            pltpu.async_copy(pt_sp.at[pidx.at[js]], wrows.at[j], psem, add=True)

        @pl.loop(0, cpw)
        def _store(j):
            pltpu.make_async_copy(wt_hbm.at[pl.ds(0, _CHUNK)], wrows.at[j], psem).wait()
            cs = pl.ds(col0 + j * _CHUNK, _CHUNK)
            pltpu.async_copy(wrows.at[j], out_hbm.at[row, cs], ssem)
            pltpu.make_async_copy(wt_hbm.at[pl.ds(0, _CHUNK)], wrows.at[j], ssem).wait()

    return k(ids, pids, word_table, pos_table)


def kernel(x_qkv, batch_size, seq_len, input_ids, position_ids, word_table, pos_table):
    return _embed_lookup(input_ids, position_ids, word_table, pos_table)


# speedup vs baseline: 1.0084x; 1.0084x over previous
import functools

import jax
import jax.numpy as jnp
from jax import lax
from jax.experimental import pallas as pl
from jax.experimental.pallas import tpu as pltpu
from jax.experimental.pallas import tpu_sc as plsc

_NC = 2
_NS = 16
_NW = _NC * _NS
_CHUNK = 64


def _embed_lookup(ids, pids, word_table, pos_table):
    b, s = ids.shape
    d = word_table.shape[1]
    n_total = b * s
    cpw = n_total // (_NW * _CHUNK)
    spw = cpw * _CHUNK
    wpb = s // spw
    mesh = plsc.VectorSubcoreMesh(core_axis_name="c", subcore_axis_name="s")

    @functools.partial(
        pl.kernel,
        out_type=jax.ShapeDtypeStruct((b, s, d), jnp.float32),
        mesh=mesh,
        scratch_types=[
            pltpu.VMEM((spw,), jnp.int32),
            pltpu.VMEM((spw,), jnp.int32),
            pltpu.VMEM((cpw, _CHUNK, d), jnp.float32),
            pltpu.VMEM_SHARED((pos_table.shape[0], d), jnp.float32),
            pltpu.SemaphoreType.DMA,
            pltpu.SemaphoreType.DMA,
            pltpu.SemaphoreType.DMA,
            pltpu.SemaphoreType.DMA,
            pltpu.SemaphoreType.DMA,
        ],
    )
    def k(ids_hbm, pids_hbm, wt_hbm, pt_hbm, out_hbm,
          widx, pidx, wrows, pt_sp, gsem, psem, isem, ssem, stsem):
        sid = lax.axis_index("s")
        wid = sid * _NC + lax.axis_index("c")
        row = wid // wpb
        col0 = (wid % wpb) * spw
        ic1 = pltpu.async_copy(ids_hbm.at[row, pl.ds(col0, spw)], widx, isem)
        ic2 = pltpu.async_copy(pids_hbm.at[row, pl.ds(col0, spw)], pidx, isem)
        prows_per_tile = pos_table.shape[0] // _NS
        pslice = pl.ds(sid * prows_per_tile, prows_per_tile)
        stage = pltpu.async_copy(pt_hbm.at[pslice], pt_sp.at[pslice], stsem)
        ic1.wait()

        @pl.loop(0, cpw)
        def _word(j):
            js = pl.ds(j * _CHUNK, _CHUNK)
            pltpu.async_copy(wt_hbm.at[widx.at[js]], wrows.at[j], gsem)

        ic2.wait()
        stage.wait()
        plsc.subcore_barrier()

        @pl.loop(0, cpw)
        def _pos(j):
            pltpu.make_async_copy(wt_hbm.at[pl.ds(0, _CHUNK)], wrows.at[j], gsem).wait()
            js = pl.ds(j * _CHUNK, _CHUNK)
            pltpu.async_copy(pt_sp.at[pidx.at[js]], wrows.at[j], psem, add=True)

        @pl.loop(0, cpw)
        def _store(j):
            pltpu.make_async_copy(wt_hbm.at[pl.ds(0, _CHUNK)], wrows.at[j], psem).wait()
            cs = pl.ds(col0 + j * _CHUNK, _CHUNK)
            pltpu.async_copy(wrows.at[j], out_hbm.at[row, cs], ssem)

        @pl.loop(0, cpw)
        def _drain(j):
            pltpu.make_async_copy(wt_hbm.at[pl.ds(0, _CHUNK)], wrows.at[j], ssem).wait()

    return k(ids, pids, word_table, pos_table)


def kernel(x_qkv, batch_size, seq_len, input_ids, position_ids, word_table, pos_table):
    return _embed_lookup(input_ids, position_ids, word_table, pos_table)
